# Initial kernel scaffold; baseline (speedup 1.0000x reference)
import jax
import jax.numpy as jnp
from jax.experimental import pallas as pl
from jax.experimental.pallas import tpu as pltpu

_TR = 128
_CORES = 2
_OUT_L = 256

_LOSSES = (
    (0, 10, 'shape_loss'),
    (1, 10, 'expression_loss'),
    (2, 9, 'global_orient_loss'),
    (3, 189, 'body_pose_loss'),
    (4, 135, 'left_hand_pose_loss'),
    (5, 135, 'right_hand_pose_loss'),
    (6, 9, 'jaw_pose_loss'),
)


def _loss_body(pb, tb, pe, te, pg, tg, pp, tp, plh, tlh, prh, trh, pj, tj,
               c_ref, o_ref, *, inv_n):
    @pl.when(pl.program_id(1) == 0)
    def _():
        o_ref[...] = jnp.zeros_like(o_ref)

    c = c_ref[...]
    lane = jax.lax.broadcasted_iota(jnp.int32, c.shape, 1)
    cf = jnp.sum(jnp.where((lane >= 67) & (lane < 118), c, 0.0),
                 axis=1, keepdims=True) * (inv_n / 51.0)
    clh = jnp.sum(jnp.where((lane >= 25) & (lane < 46), c, 0.0),
                  axis=1, keepdims=True) * (inv_n / 21.0)
    crh = jnp.sum(jnp.where((lane >= 46) & (lane < 67), c, 0.0),
                  axis=1, keepdims=True) * (inv_n / 21.0)

    def acc(row, f, p_ref, t_ref, w):
        d = p_ref[...] - t_ref[...]
        col = jnp.sum(d * d * w, axis=0, keepdims=True)
        o_ref[row:row + 1, 0:f] = o_ref[row:row + 1, 0:f] + col

    acc(0, 10, pb, tb, inv_n)
    acc(1, 10, pe, te, cf)
    acc(2, 9, pg, tg, inv_n)
    acc(3, 189, pp, tp, inv_n)
    acc(4, 135, plh, tlh, clh)
    acc(5, 135, prh, trh, crh)
    acc(6, 9, pj, tj, cf)


def kernel(stage0_betas, stage0_expression, stage0_global_orient,
           stage0_body_pose, stage0_left_hand_pose, stage0_right_hand_pose,
           stage0_jaw_pose,
           stage1_betas, stage1_expression, stage1_global_orient,
           stage1_body_pose, stage1_left_hand_pose, stage1_right_hand_pose,
           stage1_jaw_pose,
           stage2_betas, stage2_expression, stage2_global_orient,
           stage2_body_pose, stage2_left_hand_pose, stage2_right_hand_pose,
           stage2_jaw_pose,
           stage3_betas, stage3_expression, stage3_global_orient,
           stage3_body_pose, stage3_left_hand_pose, stage3_right_hand_pose,
           stage3_jaw_pose,
           tgt_conf, tgt_betas, tgt_expression, tgt_global_orient,
           tgt_body_pose, tgt_left_hand_pose, tgt_right_hand_pose,
           tgt_jaw_pose):
    b = tgt_conf.shape[0]
    steps = b // (_TR * _CORES)
    assert b % (_TR * _CORES) == 0

    def flat(x):
        return x.reshape(x.shape[0], -1).astype(jnp.float32)

    preds = (flat(stage3_betas), flat(stage3_expression),
             flat(stage3_global_orient), flat(stage3_body_pose),
             flat(stage3_left_hand_pose), flat(stage3_right_hand_pose),
             flat(stage3_jaw_pose))
    tgts = (flat(tgt_betas), flat(tgt_expression), flat(tgt_global_orient),
            flat(tgt_body_pose), flat(tgt_left_hand_pose),
            flat(tgt_right_hand_pose), flat(tgt_jaw_pose))

    def row_spec(f):
        return pl.BlockSpec((_TR, f), lambda i, r: (i * steps + r, 0))

    in_specs = []
    operands = []
    for p, t in zip(preds, tgts):
        operands += [p, t]
        in_specs += [row_spec(p.shape[1]), row_spec(t.shape[1])]
    operands.append(tgt_conf.astype(jnp.float32))
    in_specs.append(row_spec(tgt_conf.shape[1]))

    import functools
    out = pl.pallas_call(
        functools.partial(_loss_body, inv_n=1.0 / b),
        out_shape=jax.ShapeDtypeStruct((_CORES * 8, _OUT_L), jnp.float32),
        grid=(_CORES, steps),
        in_specs=in_specs,
        out_specs=pl.BlockSpec((8, _OUT_L), lambda i, r: (i, 0)),
        compiler_params=pltpu.CompilerParams(
            dimension_semantics=("parallel", "arbitrary"),
            vmem_limit_bytes=64 * 1024 * 1024),
    )(*operands)

    totals = jnp.sum(out[:8] + out[8:], axis=1)
    return {f'stage_03_{name}': totals[row] for row, _, name in _LOSSES}


# speedup vs baseline: 1.8572x; 1.8572x over previous
import functools

import jax
import jax.numpy as jnp
from jax.experimental import pallas as pl
from jax.experimental.pallas import tpu as pltpu

_TR = 128
_CORES = 2
_OUT_L = 256

_LOSSES = (
    (0, 10, 'shape_loss'),
    (1, 10, 'expression_loss'),
    (2, 9, 'global_orient_loss'),
    (3, 189, 'body_pose_loss'),
    (4, 135, 'left_hand_pose_loss'),
    (5, 135, 'right_hand_pose_loss'),
    (6, 9, 'jaw_pose_loss'),
)


def _loss_body(pb, tb, pe, te, pg, tg, pp, tp, plh, tlh, prh, trh, pj, tj,
               c_ref, o_ref, *, inv_n):
    @pl.when(pl.program_id(1) == 0)
    def _():
        o_ref[...] = jnp.zeros_like(o_ref)

    c = c_ref[...]
    lane = jax.lax.broadcasted_iota(jnp.int32, c.shape, 1)
    cf = jnp.sum(jnp.where((lane >= 67) & (lane < 118), c, 0.0),
                 axis=1, keepdims=True) * (inv_n / 51.0)
    clh = jnp.sum(jnp.where((lane >= 25) & (lane < 46), c, 0.0),
                  axis=1, keepdims=True) * (inv_n / 21.0)
    crh = jnp.sum(jnp.where((lane >= 46) & (lane < 67), c, 0.0),
                  axis=1, keepdims=True) * (inv_n / 21.0)

    def acc(row, f, p_ref, t_ref, w):
        d = p_ref[...] - t_ref[...]
        col = jnp.sum(d * d * w, axis=0, keepdims=True)
        o_ref[row:row + 1, 0:f] = o_ref[row:row + 1, 0:f] + col

    acc(0, 10, pb, tb, inv_n)
    acc(1, 10, pe, te, cf)
    acc(2, 9, pg, tg, inv_n)
    acc(3, 189, pp, tp, inv_n)
    acc(4, 135, plh, tlh, clh)
    acc(5, 135, prh, trh, crh)
    acc(6, 9, pj, tj, cf)


def kernel(stage0_betas, stage0_expression, stage0_global_orient,
           stage0_body_pose, stage0_left_hand_pose, stage0_right_hand_pose,
           stage0_jaw_pose,
           stage1_betas, stage1_expression, stage1_global_orient,
           stage1_body_pose, stage1_left_hand_pose, stage1_right_hand_pose,
           stage1_jaw_pose,
           stage2_betas, stage2_expression, stage2_global_orient,
           stage2_body_pose, stage2_left_hand_pose, stage2_right_hand_pose,
           stage2_jaw_pose,
           stage3_betas, stage3_expression, stage3_global_orient,
           stage3_body_pose, stage3_left_hand_pose, stage3_right_hand_pose,
           stage3_jaw_pose,
           tgt_conf, tgt_betas, tgt_expression, tgt_global_orient,
           tgt_body_pose, tgt_left_hand_pose, tgt_right_hand_pose,
           tgt_jaw_pose):
    b = tgt_conf.shape[0]
    steps = b // (_TR * _CORES)
    assert b % (_TR * _CORES) == 0

    def flat(x):
        return x.reshape(x.shape[0], -1).astype(jnp.float32)

    preds = (flat(stage3_betas), flat(stage3_expression),
             flat(stage3_global_orient), flat(stage3_body_pose),
             flat(stage3_left_hand_pose), flat(stage3_right_hand_pose),
             flat(stage3_jaw_pose))
    tgts = (flat(tgt_betas), flat(tgt_expression), flat(tgt_global_orient),
            flat(tgt_body_pose), flat(tgt_left_hand_pose),
            flat(tgt_right_hand_pose), flat(tgt_jaw_pose))

    def row_spec(f):
        return pl.BlockSpec((_TR, f), lambda i, r: (i * steps + r, 0))

    in_specs = []
    operands = []
    for p, t in zip(preds, tgts):
        operands += [p, t]
        in_specs += [row_spec(p.shape[1]), row_spec(t.shape[1])]
    operands.append(tgt_conf.astype(jnp.float32))
    in_specs.append(row_spec(tgt_conf.shape[1]))

    out = pl.pallas_call(
        functools.partial(_loss_body, inv_n=1.0 / b),
        out_shape=jax.ShapeDtypeStruct((_CORES * 8, _OUT_L), jnp.float32),
        grid=(_CORES, steps),
        in_specs=in_specs,
        out_specs=pl.BlockSpec((8, _OUT_L), lambda i, r: (i, 0)),
        compiler_params=pltpu.CompilerParams(
            dimension_semantics=("parallel", "arbitrary"),
            vmem_limit_bytes=64 * 1024 * 1024),
    )(*operands)

    totals = jnp.sum(out[:8] + out[8:], axis=1)
    return {f'stage_03_{name}': totals[row] for row, _, name in _LOSSES}


# speedup vs baseline: 2.0606x; 1.1095x over previous
import functools

import jax
import jax.numpy as jnp
from jax.experimental import pallas as pl
from jax.experimental.pallas import tpu as pltpu

_TR = 512
_CORES = 2
_OUT_L = 256

_LOSSES = (
    (0, 10, 'shape_loss'),
    (1, 10, 'expression_loss'),
    (2, 9, 'global_orient_loss'),
    (3, 189, 'body_pose_loss'),
    (4, 135, 'left_hand_pose_loss'),
    (5, 135, 'right_hand_pose_loss'),
    (6, 9, 'jaw_pose_loss'),
)


def _loss_body(pb, tb, pe, te, pg, tg, pp, tp, plh, tlh, prh, trh, pj, tj,
               c_ref, o_ref, *, inv_n):
    @pl.when(pl.program_id(1) == 0)
    def _():
        o_ref[...] = jnp.zeros_like(o_ref)

    c = c_ref[...]
    lane = jax.lax.broadcasted_iota(jnp.int32, c.shape, 1)
    cf = jnp.sum(jnp.where((lane >= 67) & (lane < 118), c, 0.0),
                 axis=1, keepdims=True) * (inv_n / 51.0)
    clh = jnp.sum(jnp.where((lane >= 25) & (lane < 46), c, 0.0),
                  axis=1, keepdims=True) * (inv_n / 21.0)
    crh = jnp.sum(jnp.where((lane >= 46) & (lane < 67), c, 0.0),
                  axis=1, keepdims=True) * (inv_n / 21.0)

    def acc(row, f, p_ref, t_ref, w):
        d = p_ref[...] - t_ref[...]
        col = jnp.sum(d * d * w, axis=0, keepdims=True)
        o_ref[row:row + 1, 0:f] = o_ref[row:row + 1, 0:f] + col

    acc(0, 10, pb, tb, inv_n)
    acc(1, 10, pe, te, cf)
    acc(2, 9, pg, tg, inv_n)
    acc(3, 189, pp, tp, inv_n)
    acc(4, 135, plh, tlh, clh)
    acc(5, 135, prh, trh, crh)
    acc(6, 9, pj, tj, cf)


def kernel(stage0_betas, stage0_expression, stage0_global_orient,
           stage0_body_pose, stage0_left_hand_pose, stage0_right_hand_pose,
           stage0_jaw_pose,
           stage1_betas, stage1_expression, stage1_global_orient,
           stage1_body_pose, stage1_left_hand_pose, stage1_right_hand_pose,
           stage1_jaw_pose,
           stage2_betas, stage2_expression, stage2_global_orient,
           stage2_body_pose, stage2_left_hand_pose, stage2_right_hand_pose,
           stage2_jaw_pose,
           stage3_betas, stage3_expression, stage3_global_orient,
           stage3_body_pose, stage3_left_hand_pose, stage3_right_hand_pose,
           stage3_jaw_pose,
           tgt_conf, tgt_betas, tgt_expression, tgt_global_orient,
           tgt_body_pose, tgt_left_hand_pose, tgt_right_hand_pose,
           tgt_jaw_pose):
    b = tgt_conf.shape[0]
    steps = b // (_TR * _CORES)
    assert b % (_TR * _CORES) == 0

    def flat(x):
        return x.reshape(x.shape[0], -1).astype(jnp.float32)

    preds = (flat(stage3_betas), flat(stage3_expression),
             flat(stage3_global_orient), flat(stage3_body_pose),
             flat(stage3_left_hand_pose), flat(stage3_right_hand_pose),
             flat(stage3_jaw_pose))
    tgts = (flat(tgt_betas), flat(tgt_expression), flat(tgt_global_orient),
            flat(tgt_body_pose), flat(tgt_left_hand_pose),
            flat(tgt_right_hand_pose), flat(tgt_jaw_pose))

    def row_spec(f):
        return pl.BlockSpec((_TR, f), lambda i, r: (i * steps + r, 0))

    in_specs = []
    operands = []
    for p, t in zip(preds, tgts):
        operands += [p, t]
        in_specs += [row_spec(p.shape[1]), row_spec(t.shape[1])]
    operands.append(tgt_conf.astype(jnp.float32))
    in_specs.append(row_spec(tgt_conf.shape[1]))

    out = pl.pallas_call(
        functools.partial(_loss_body, inv_n=1.0 / b),
        out_shape=jax.ShapeDtypeStruct((_CORES * 8, _OUT_L), jnp.float32),
        grid=(_CORES, steps),
        in_specs=in_specs,
        out_specs=pl.BlockSpec((8, _OUT_L), lambda i, r: (i, 0)),
        compiler_params=pltpu.CompilerParams(
            dimension_semantics=("parallel", "arbitrary"),
            vmem_limit_bytes=64 * 1024 * 1024),
    )(*operands)

    totals = jnp.sum(out[:8] + out[8:], axis=1)
    return {f'stage_03_{name}': totals[row] for row, _, name in _LOSSES}
